# baseline (speedup 1.0000x reference)
import math
import functools

import jax
import jax.numpy as jnp
from jax import lax
from jax.experimental import pallas as pl
from jax.experimental.pallas import tpu as pltpu
from jax.experimental.pallas import tpu_sc as plsc

H_R, W_R = 64, 2048
H_B, W_B = 512, 512
R_MIN, R_MAX = 2.0, 50.0
R_BINS = 30
PHI_MIN, PHI_MAX = -math.pi, math.pi
THETA_MIN, THETA_MAX = math.radians(-25.0), math.radians(3.0)
XMIN, XMAX, YMIN, YMAX = -50.0, 50.0, -50.0, 50.0
BATCH, CHAN = 2, 64
P = H_R * W_R
N = H_B * W_B
QUARTER = N // 4

NUM_CORES = 2
NUM_SUBCORES = 16
NUM_WORKERS = NUM_CORES * NUM_SUBCORES
LANES = 16
VECS_PER_ROW = W_R // LANES
LANE_STRIDE = W_R // LANES


def _cell_body(bin_ref, table_ref, cell_ref):
    b = bin_ref[...]
    res = jnp.full((H_R, W_R), -1, jnp.int32)
    for k in range(R_BINS):
        res = jnp.where(b == k, table_ref[pl.ds(k * H_R, H_R), :], res)
    cell_ref[...] = res


def _compute_cells(bin2d, table):
    return pl.pallas_call(
        _cell_body,
        grid=(BATCH,),
        in_specs=[
            pl.BlockSpec((H_R, W_R), lambda i: (i, 0)),
            pl.BlockSpec((R_BINS * H_R, W_R), lambda i: (0, 0)),
        ],
        out_specs=pl.BlockSpec((H_R, W_R), lambda i: (i, 0)),
        out_shape=jax.ShapeDtypeStruct((BATCH * H_R, W_R), jnp.int32),
    )(bin2d, table)


def _sc_body(feat_hbm, cell_hbm, out_hbm, cell_row, feat_row, acc):
    wid = lax.axis_index("s") * NUM_CORES + lax.axis_index("c")

    def task_loop(t, carry):
        task = wid + NUM_WORKERS * t
        b = task // CHAN
        c = lax.rem(task, CHAN)

        def quarter_loop(q, carry):
            base = q * QUARTER

            def zero_loop(i, carry):
                acc[pl.ds(i * LANES, LANES)] = jnp.zeros((LANES,), jnp.float32)
                return carry

            lax.fori_loop(0, QUARTER // LANES, zero_loop, 0)

            def row_loop(h, carry):
                pltpu.sync_copy(cell_hbm.at[b, pl.ds(h * W_R, W_R)], cell_row)
                pltpu.sync_copy(feat_hbm.at[b, c, pl.ds(h * W_R, W_R)], feat_row)

                def vec_loop(j, carry):
                    gidx = j + lax.iota(jnp.int32, LANES) * LANE_STRIDE
                    cell = plsc.load_gather(cell_row, [gidx])
                    f = plsc.load_gather(feat_row, [gidx])
                    loc = cell - base
                    m = (loc >= 0) & (loc < QUARTER)
                    locc = jnp.minimum(jnp.maximum(loc, 0), QUARTER - 1)
                    old = plsc.load_gather(acc, [locc])
                    plsc.store_scatter(acc, [locc], jnp.maximum(old, f), mask=m)
                    return carry

                lax.fori_loop(0, VECS_PER_ROW, vec_loop, 0)
                return carry

            lax.fori_loop(0, H_R, row_loop, 0)
            pltpu.sync_copy(acc, out_hbm.at[b, c, pl.ds(base, QUARTER)])
            return carry

        lax.fori_loop(0, 4, quarter_loop, 0)
        return carry

    lax.fori_loop(0, (BATCH * CHAN) // NUM_WORKERS, task_loop, 0)


def _sc_scatter_max(feat_flat, cells):
    mesh = plsc.VectorSubcoreMesh(core_axis_name="c", subcore_axis_name="s")
    kfn = functools.partial(
        pl.kernel,
        mesh=mesh,
        out_type=jax.ShapeDtypeStruct((BATCH, CHAN, N), jnp.float32),
        scratch_types=[
            pltpu.VMEM((W_R,), jnp.int32),
            pltpu.VMEM((W_R,), jnp.float32),
            pltpu.VMEM((QUARTER,), jnp.float32),
        ],
        compiler_params=pltpu.CompilerParams(needs_layout_passes=False),
    )(_sc_body)
    return kfn(feat_flat, cells)


def kernel(rv_feat, rv_range_bin):
    bin2d = rv_range_bin.astype(jnp.int32).reshape(BATCH * H_R, W_R)
    dr = (R_MAX - R_MIN) / R_BINS
    r = jnp.arange(R_BINS, dtype=jnp.float32).reshape(R_BINS, 1, 1) * dr + (R_MIN + dr / 2.0)
    theta = jnp.linspace(THETA_MAX, THETA_MIN, H_R, dtype=jnp.float32).reshape(1, H_R, 1)
    phi = jnp.linspace(PHI_MIN, PHI_MAX, W_R, dtype=jnp.float32).reshape(1, 1, W_R)
    x = r * jnp.cos(theta) * jnp.cos(phi)
    y = r * jnp.cos(theta) * jnp.sin(phi)
    u = ((x - XMIN) / (XMAX - XMIN) * (W_B - 1)).astype(jnp.int32)
    v = ((YMAX - y) / (YMAX - YMIN) * (H_B - 1)).astype(jnp.int32)
    valid = (u >= 0) & (u < W_B) & (v >= 0) & (v < H_B)
    table = jnp.where(valid, v * W_B + u, -1).reshape(R_BINS * H_R, W_R)

    cells = _compute_cells(bin2d, table).reshape(BATCH, P)
    feat_flat = rv_feat.reshape(BATCH, CHAN, P)
    out = _sc_scatter_max(feat_flat, cells)
    return out.reshape(BATCH, CHAN, H_B, W_B)
